# Initial kernel scaffold; baseline (speedup 1.0000x reference)
import jax
import jax.numpy as jnp
from jax.experimental import pallas as pl


def kernel(x, W, b, gate_bias):
    raise NotImplementedError("write your pallas kernel here")


# speedup vs baseline: 1.5686x; 1.5686x over previous
import functools

import jax
import jax.numpy as jnp
from jax.experimental import pallas as pl
from jax.experimental.pallas import tpu as pltpu

_NUM_EXPERTS = 64
_TOPK = 8
_TEMP = 1.0
_HIDDEN = 4096
_BM = 512


def _router_block(x_ref, wt_ref, b_ref, gb_ref, out_ref, ids_ref):
    x = x_ref[...]
    wt = wt_ref[...]
    gate = jax.lax.dot_general(
        x, wt, (((1,), (0,)), ((), ())),
        preferred_element_type=jnp.float32,
    )
    gate = gate * (1.0 / _TEMP) + b_ref[...]
    work = gate + gb_ref[...]
    lane = jax.lax.broadcasted_iota(jnp.int32, gate.shape, 1)

    selected = jnp.zeros(gate.shape, jnp.bool_)
    ids_cols = []
    for _ in range(_TOPK):
        m = jnp.max(work, axis=1, keepdims=True)
        eq = work == m
        idx = jnp.min(jnp.where(eq, lane, _NUM_EXPERTS), axis=1, keepdims=True)
        onehot = lane == idx
        selected = jnp.logical_or(selected, onehot)
        work = jnp.where(onehot, -jnp.inf, work)
        ids_cols.append(idx)
    ids_ref[...] = jnp.concatenate(ids_cols, axis=1)

    masked = jnp.where(selected, gate, -jnp.inf)
    mx = jnp.max(masked, axis=1, keepdims=True)
    e = jnp.where(selected, jnp.exp(masked - mx), 0.0)
    out_ref[...] = e / jnp.sum(e, axis=1, keepdims=True)


@functools.partial(jax.jit, static_argnames=())
def kernel(x, W, b, gate_bias):
    B, S, H = x.shape
    M = B * S
    x2 = x.reshape(M, H)
    wt = W.T
    b2 = b.reshape(1, _NUM_EXPERTS)
    gb2 = gate_bias.reshape(1, _NUM_EXPERTS)

    grid = (M // _BM,)
    out, ids = pl.pallas_call(
        _router_block,
        grid=grid,
        in_specs=[
            pl.BlockSpec((_BM, H), lambda i: (i, 0)),
            pl.BlockSpec((H, _NUM_EXPERTS), lambda i: (0, 0)),
            pl.BlockSpec((1, _NUM_EXPERTS), lambda i: (0, 0)),
            pl.BlockSpec((1, _NUM_EXPERTS), lambda i: (0, 0)),
        ],
        out_specs=[
            pl.BlockSpec((_BM, _NUM_EXPERTS), lambda i: (i, 0)),
            pl.BlockSpec((_BM, _TOPK), lambda i: (i, 0)),
        ],
        out_shape=[
            jax.ShapeDtypeStruct((M, _NUM_EXPERTS), jnp.float32),
            jax.ShapeDtypeStruct((M, _TOPK), jnp.int32),
        ],
        compiler_params=pltpu.CompilerParams(
            dimension_semantics=("arbitrary",),
        ),
    )(x2, wt, b2, gb2)
    return out.reshape(B, S, _NUM_EXPERTS), ids.reshape(B, S, _TOPK)


# speedup vs baseline: 1.7528x; 1.1174x over previous
import functools

import jax
import jax.numpy as jnp
from jax.experimental import pallas as pl
from jax.experimental.pallas import tpu as pltpu

_NUM_EXPERTS = 64
_TOPK = 8
_TEMP = 1.0
_HIDDEN = 4096
_BM = 512


def _router_block(x_ref, wt_ref, b_ref, gb_ref, out_ref, ids_ref):
    x = x_ref[...]
    wt = wt_ref[...]
    gate = jax.lax.dot_general(
        x, wt, (((1,), (0,)), ((), ())),
        preferred_element_type=jnp.float32,
    )
    gate = gate * (1.0 / _TEMP) + b_ref[...]
    work = gate + gb_ref[...]
    lane = jax.lax.broadcasted_iota(jnp.int32, gate.shape, 1)

    bits = jax.lax.bitcast_convert_type(work, jnp.int32)
    skey = bits ^ ((bits >> 31) & jnp.int32(0x7FFFFFFF))
    key = (skey & jnp.int32(-64)) | (jnp.int32(_NUM_EXPERTS - 1) - lane)

    sentinel = jnp.int32(-(2 ** 31))
    ids_cols = []
    for _ in range(_TOPK):
        m = jnp.max(key, axis=1, keepdims=True)
        ids_cols.append(jnp.int32(_NUM_EXPERTS - 1) - (m & jnp.int32(63)))
        key = jnp.where(key == m, sentinel, key)
    ids_ref[...] = jnp.concatenate(ids_cols, axis=1)

    selected = key == sentinel
    e = jnp.where(selected, jnp.exp(gate), 0.0)
    out_ref[...] = e / jnp.sum(e, axis=1, keepdims=True)


@functools.partial(jax.jit, static_argnames=())
def kernel(x, W, b, gate_bias):
    B, S, H = x.shape
    M = B * S
    x2 = x.reshape(M, H)
    wt = W.T
    b2 = b.reshape(1, _NUM_EXPERTS)
    gb2 = gate_bias.reshape(1, _NUM_EXPERTS)

    grid = (M // _BM,)
    out, ids = pl.pallas_call(
        _router_block,
        grid=grid,
        in_specs=[
            pl.BlockSpec((_BM, H), lambda i: (i, 0)),
            pl.BlockSpec((H, _NUM_EXPERTS), lambda i: (0, 0)),
            pl.BlockSpec((1, _NUM_EXPERTS), lambda i: (0, 0)),
            pl.BlockSpec((1, _NUM_EXPERTS), lambda i: (0, 0)),
        ],
        out_specs=[
            pl.BlockSpec((_BM, _NUM_EXPERTS), lambda i: (i, 0)),
            pl.BlockSpec((_BM, _TOPK), lambda i: (i, 0)),
        ],
        out_shape=[
            jax.ShapeDtypeStruct((M, _NUM_EXPERTS), jnp.float32),
            jax.ShapeDtypeStruct((M, _TOPK), jnp.int32),
        ],
        compiler_params=pltpu.CompilerParams(
            dimension_semantics=("arbitrary",),
        ),
    )(x2, wt, b2, gb2)
    return out.reshape(B, S, _NUM_EXPERTS), ids.reshape(B, S, _TOPK)


# speedup vs baseline: 2.2586x; 1.2886x over previous
import functools

import jax
import jax.numpy as jnp
from jax.experimental import pallas as pl
from jax.experimental.pallas import tpu as pltpu

_NUM_EXPERTS = 64
_TOPK = 8
_TEMP = 1.0
_HIDDEN = 4096
_BM = 512


def _router_block(x_ref, w_ref, b_ref, gb_ref, out_ref, ids_ref):
    x = x_ref[...]
    w = w_ref[...]
    gate = jax.lax.dot_general(
        w, x, (((1,), (1,)), ((), ())),
        preferred_element_type=jnp.float32,
    )
    gate = gate * (1.0 / _TEMP) + b_ref[...]
    work = gate + gb_ref[...]
    row = jax.lax.broadcasted_iota(jnp.int32, gate.shape, 0)

    bits = jax.lax.bitcast_convert_type(work, jnp.int32)
    skey = bits ^ ((bits >> 31) & jnp.int32(0x7FFFFFFF))
    key = (skey & jnp.int32(-64)) | (jnp.int32(_NUM_EXPERTS - 1) - row)

    sentinel = jnp.int32(-(2 ** 31))
    ids_rows = []
    for _ in range(_TOPK):
        m = jnp.max(key, axis=0, keepdims=True)
        ids_rows.append(jnp.int32(_NUM_EXPERTS - 1) - (m & jnp.int32(63)))
        key = jnp.where(key == m, sentinel, key)
    ids_ref[...] = jnp.concatenate(ids_rows, axis=0)

    selected = key == sentinel
    e = jnp.where(selected, jnp.exp(gate), 0.0)
    out_ref[...] = e / jnp.sum(e, axis=0, keepdims=True)


@functools.partial(jax.jit, static_argnames=())
def kernel(x, W, b, gate_bias):
    B, S, H = x.shape
    M = B * S
    x2 = x.reshape(M, H)
    b2 = b.reshape(_NUM_EXPERTS, 1)
    gb2 = gate_bias.reshape(_NUM_EXPERTS, 1)

    grid = (M // _BM,)
    out_t, ids_t = pl.pallas_call(
        _router_block,
        grid=grid,
        in_specs=[
            pl.BlockSpec((_BM, H), lambda i: (i, 0)),
            pl.BlockSpec((_NUM_EXPERTS, H), lambda i: (0, 0)),
            pl.BlockSpec((_NUM_EXPERTS, 1), lambda i: (0, 0)),
            pl.BlockSpec((_NUM_EXPERTS, 1), lambda i: (0, 0)),
        ],
        out_specs=[
            pl.BlockSpec((_NUM_EXPERTS, _BM), lambda i: (0, i)),
            pl.BlockSpec((_TOPK, _BM), lambda i: (0, i)),
        ],
        out_shape=[
            jax.ShapeDtypeStruct((_NUM_EXPERTS, M), jnp.float32),
            jax.ShapeDtypeStruct((_TOPK, M), jnp.int32),
        ],
        compiler_params=pltpu.CompilerParams(
            dimension_semantics=("arbitrary",),
        ),
    )(x2, W, b2, gb2)
    out = out_t.T.reshape(B, S, _NUM_EXPERTS)
    ids = ids_t.T.reshape(B, S, _TOPK)
    return out, ids
